# Initial kernel scaffold; baseline (speedup 1.0000x reference)
import jax
import jax.numpy as jnp
from jax.experimental import pallas as pl


def kernel(batch, weight):
    raise NotImplementedError("write your pallas kernel here")


# speedup vs baseline: 1.2693x; 1.2693x over previous
import functools

import jax
import jax.numpy as jnp
from jax import lax
from jax.experimental import pallas as pl
from jax.experimental.pallas import tpu as pltpu
from jax.experimental.pallas import tpu_sc as plsc

VOCAB = 1000000
EMBED_DIM = 32
B = 16384
L = 50

NC = 2
NS = 16
NW = NC * NS

TOTAL = B * L
B_PER_W = TOTAL // NW
G = 10
CH = G * 128
NCHUNK = B_PER_W // CH


def _emb_body(idx_hbm, table_hbm, out_hbm, idx_v, rows_v, sem):
  c = lax.axis_index("c")
  s = lax.axis_index("s")
  wid = s * NC + c

  def chunk_body(i):
    pltpu.sync_copy(idx_hbm.at[wid, i], idx_v)
    copies = []
    for j in range(G):
      copies.append(
          pltpu.async_copy(
              table_hbm.at[idx_v.at[j]],
              rows_v.at[pl.ds(j * 128, 128)],
              sem,
          )
      )
    for cp in copies:
      cp.wait()
    pltpu.sync_copy(rows_v, out_hbm.at[wid, i])

  pl.loop(0, NCHUNK)(chunk_body)


@jax.jit
def _embedding_sc(batch, weight):
  idx = batch.reshape(NW, NCHUNK, G, 128)
  mesh = plsc.VectorSubcoreMesh(core_axis_name="c", subcore_axis_name="s")
  out = pl.kernel(
      _emb_body,
      out_type=jax.ShapeDtypeStruct((NW, NCHUNK, CH, EMBED_DIM), jnp.float32),
      mesh=mesh,
      scratch_types=[
          pltpu.VMEM((G, 128), jnp.int32),
          pltpu.VMEM((CH, EMBED_DIM), jnp.float32),
          pltpu.SemaphoreType.DMA,
      ],
      compiler_params=pltpu.CompilerParams(use_tc_tiling_on_sc=False),
  )(idx, weight)
  return out.reshape(B, L, EMBED_DIM)


def kernel(batch, weight):
  return _embedding_sc(batch, weight)


# speedup vs baseline: 1.2858x; 1.0130x over previous
import functools

import jax
import jax.numpy as jnp
from jax import lax
from jax.experimental import pallas as pl
from jax.experimental.pallas import tpu as pltpu
from jax.experimental.pallas import tpu_sc as plsc

VOCAB = 1000000
EMBED_DIM = 32
B = 16384
L = 50

NC = 2
NS = 16
NW = NC * NS

TOTAL = B * L
B_PER_W = TOTAL // NW
G = 10
CH = G * 128
NCHUNK = B_PER_W // CH


def _emb_body(idx_hbm, table_hbm, out_hbm, idx_v, rows_v,
              gsem0, gsem1, ssem0, ssem1):
  c = lax.axis_index("c")
  s = lax.axis_index("s")
  wid = s * NC + c
  gsems = (gsem0, gsem1)
  ssems = (ssem0, ssem1)

  def fire(i, b):
    pltpu.sync_copy(idx_hbm.at[wid, i], idx_v.at[b])
    for j in range(G):
      pltpu.async_copy(
          table_hbm.at[idx_v.at[b, j]],
          rows_v.at[b, pl.ds(j * 128, 128)],
          gsems[b],
      )

  def drain(b):
    for j in range(G):
      pltpu.make_async_copy(
          table_hbm.at[idx_v.at[b, j]],
          rows_v.at[b, pl.ds(j * 128, 128)],
          gsems[b],
      ).wait()

  def store(i, b):
    pltpu.async_copy(rows_v.at[b], out_hbm.at[wid, i], ssems[b])

  def wait_store(i, b):
    pltpu.make_async_copy(rows_v.at[b], out_hbm.at[wid, i], ssems[b]).wait()

  fire(0, 0)
  fire(1, 1)

  def outer(i):
    drain(0)
    store(i, 0)

    @pl.when(i + 2 < NCHUNK)
    def _():
      wait_store(i, 0)
      fire(i + 2, 0)

    drain(1)
    store(i + 1, 1)

    @pl.when(i + 3 < NCHUNK)
    def _():
      wait_store(i + 1, 1)
      fire(i + 3, 1)

  pl.loop(0, NCHUNK, step=2)(outer)
  wait_store(NCHUNK - 2, 0)
  wait_store(NCHUNK - 1, 1)


@jax.jit
def _embedding_sc(batch, weight):
  idx = batch.reshape(NW, NCHUNK, G, 128)
  mesh = plsc.VectorSubcoreMesh(core_axis_name="c", subcore_axis_name="s")
  out = pl.kernel(
      _emb_body,
      out_type=jax.ShapeDtypeStruct((NW, NCHUNK, CH, EMBED_DIM), jnp.float32),
      mesh=mesh,
      scratch_types=[
          pltpu.VMEM((2, G, 128), jnp.int32),
          pltpu.VMEM((2, CH, EMBED_DIM), jnp.float32),
          pltpu.SemaphoreType.DMA,
          pltpu.SemaphoreType.DMA,
          pltpu.SemaphoreType.DMA,
          pltpu.SemaphoreType.DMA,
      ],
      compiler_params=pltpu.CompilerParams(use_tc_tiling_on_sc=False),
  )(idx, weight)
  return out.reshape(B, L, EMBED_DIM)


def kernel(batch, weight):
  return _embedding_sc(batch, weight)
